# baseline (speedup 1.0000x reference)
import functools

import jax
import jax.numpy as jnp
from jax import lax
from jax.experimental import pallas as pl
from jax.experimental.pallas import tpu as pltpu
from jax.experimental.pallas import tpu_sc as plsc

NC = 2
NS = 16
NW = NC * NS
CH = 64
K = 5
DP = 128


def kernel(token_ids, weight):
    B, H = token_ids.shape
    V, D = weight.shape
    N = B * H
    per_w = N // NW
    n_ch = per_w // CH
    n_g = n_ch // K
    G = K * CH
    assert per_w * NW == N and n_g * K == n_ch and n_g % 2 == 0

    idx = token_ids.reshape(NW, n_ch, CH).astype(jnp.int32)
    wp = jnp.pad(weight, ((0, 0), (0, DP - D)))
    mesh = plsc.VectorSubcoreMesh(core_axis_name="c", subcore_axis_name="s")

    @functools.partial(
        pl.kernel,
        out_type=jax.ShapeDtypeStruct((N, DP), jnp.float32),
        mesh=mesh,
        scratch_types=[
            pltpu.VMEM((n_ch, CH), jnp.int32),
            pltpu.VMEM((2, G, DP), jnp.float32),
            pltpu.SemaphoreType.DMA,
            pltpu.SemaphoreType.DMA,
            pltpu.SemaphoreType.DMA,
            pltpu.SemaphoreType.DMA,
        ],
    )
    def gather_kernel(idx_hbm, tab_hbm, out_hbm, idx_v, rows_v, g0, g1, o0, o1):
        wid = lax.axis_index("s") * NC + lax.axis_index("c")
        base = wid * per_w
        pltpu.sync_copy(idx_hbm.at[wid], idx_v)

        def fire_gathers(t, s, sem):
            for i in range(K):
                pltpu.async_copy(
                    tab_hbm.at[idx_v.at[t * K + i]],
                    rows_v.at[s, pl.ds(i * CH, CH)],
                    sem,
                )

        def fire_out(t, s, sem):
            pltpu.async_copy(rows_v.at[s], out_hbm.at[pl.ds(base + t * G, G)], sem)

        def drain_gathers(s, sem):
            pltpu.make_async_copy(tab_hbm.at[pl.ds(0, G)], rows_v.at[s], sem).wait()

        def drain_out(s, sem):
            pltpu.make_async_copy(rows_v.at[s], out_hbm.at[pl.ds(base, G)], sem).wait()

        fire_gathers(0, 0, g0)
        fire_gathers(1, 1, g1)

        def body(u, carry):
            t0 = 2 * u
            drain_gathers(0, g0)
            fire_out(t0 - 2, 0, o0)
            drain_out(0, o0)
            fire_gathers(t0, 0, g0)
            drain_gathers(1, g1)
            fire_out(t0 - 1, 1, o1)
            drain_out(1, o1)
            fire_gathers(t0 + 1, 1, g1)
            return carry

        lax.fori_loop(1, n_g // 2, body, 0)

        drain_gathers(0, g0)
        fire_out(n_g - 2, 0, o0)
        drain_gathers(1, g1)
        fire_out(n_g - 1, 1, o1)
        drain_out(0, o0)
        drain_out(1, o1)

    out = gather_kernel(idx, wp)
    return out[:, :D].reshape(B, H, D)


# speedup vs baseline: 1.0320x; 1.0320x over previous
import functools

import jax
import jax.numpy as jnp
from jax import lax
from jax.experimental import pallas as pl
from jax.experimental.pallas import tpu as pltpu
from jax.experimental.pallas import tpu_sc as plsc

NC = 2
NS = 16
NW = NC * NS
CH = 128
K = 5


def kernel(token_ids, weight):
    B, H = token_ids.shape
    V, D = weight.shape
    N = B * H
    per_w = N // NW
    n_ch = per_w // CH
    n_g = n_ch // K
    G = K * CH
    assert per_w * NW == N and n_g * K == n_ch and n_g % 2 == 0

    idx = token_ids.reshape(NW, n_ch, CH).astype(jnp.int32)
    mesh = plsc.VectorSubcoreMesh(core_axis_name="c", subcore_axis_name="s")

    @functools.partial(
        pl.kernel,
        out_type=jax.ShapeDtypeStruct((N, D), jnp.float32),
        mesh=mesh,
        scratch_types=[
            pltpu.VMEM((n_ch, CH), jnp.int32),
            pltpu.VMEM((2, G, D), jnp.float32),
            pltpu.SemaphoreType.DMA,
            pltpu.SemaphoreType.DMA,
            pltpu.SemaphoreType.DMA,
            pltpu.SemaphoreType.DMA,
        ],
        compiler_params=pltpu.CompilerParams(use_tc_tiling_on_sc=False),
    )
    def gather_kernel(idx_hbm, tab_hbm, out_hbm, idx_v, rows_v, g0, g1, o0, o1):
        wid = lax.axis_index("s") * NC + lax.axis_index("c")
        base = wid * per_w
        pltpu.sync_copy(idx_hbm.at[wid], idx_v)

        def fire_gathers(t, s, sem):
            for i in range(K):
                pltpu.async_copy(
                    tab_hbm.at[idx_v.at[t * K + i]],
                    rows_v.at[s, pl.ds(i * CH, CH)],
                    sem,
                )

        def fire_out(t, s, sem):
            pltpu.async_copy(rows_v.at[s], out_hbm.at[pl.ds(base + t * G, G)], sem)

        def drain_gathers(s, sem):
            pltpu.make_async_copy(tab_hbm.at[pl.ds(0, G)], rows_v.at[s], sem).wait()

        def drain_out(s, sem):
            pltpu.make_async_copy(rows_v.at[s], out_hbm.at[pl.ds(base, G)], sem).wait()

        fire_gathers(0, 0, g0)
        fire_gathers(1, 1, g1)

        def body(u, carry):
            t0 = 2 * u
            drain_gathers(0, g0)
            fire_out(t0 - 2, 0, o0)
            drain_out(0, o0)
            fire_gathers(t0, 0, g0)
            drain_gathers(1, g1)
            fire_out(t0 - 1, 1, o1)
            drain_out(1, o1)
            fire_gathers(t0 + 1, 1, g1)
            return carry

        lax.fori_loop(1, n_g // 2, body, 0)

        drain_gathers(0, g0)
        fire_out(n_g - 2, 0, o0)
        drain_gathers(1, g1)
        fire_out(n_g - 1, 1, o1)
        drain_out(0, o0)
        drain_out(1, o1)

    out = gather_kernel(idx, weight)
    return out.reshape(B, H, D)


# speedup vs baseline: 1.0328x; 1.0008x over previous
import functools

import jax
import jax.numpy as jnp
from jax import lax
from jax.experimental import pallas as pl
from jax.experimental.pallas import tpu as pltpu
from jax.experimental.pallas import tpu_sc as plsc

NC = 2
NS = 16
NW = NC * NS
CH = 128
K = 5


def kernel(token_ids, weight):
    B, H = token_ids.shape
    V, D = weight.shape
    N = B * H
    per_w = N // NW
    n_ch = per_w // CH
    n_g = n_ch // K
    G = K * CH
    GB = G // H
    per_wb = per_w // H
    assert per_w * NW == N and n_g * K == n_ch and n_g % 2 == 0
    assert GB * H == G and per_wb * H == per_w

    idx = token_ids.reshape(NW, n_ch, CH).astype(jnp.int32)
    mesh = plsc.VectorSubcoreMesh(core_axis_name="c", subcore_axis_name="s")

    @functools.partial(
        pl.kernel,
        out_type=jax.ShapeDtypeStruct((B, H, D), jnp.float32),
        mesh=mesh,
        scratch_types=[
            pltpu.VMEM((n_ch, CH), jnp.int32),
            pltpu.VMEM((2, G, D), jnp.float32),
            pltpu.SemaphoreType.DMA,
            pltpu.SemaphoreType.DMA,
            pltpu.SemaphoreType.DMA,
            pltpu.SemaphoreType.DMA,
        ],
        compiler_params=pltpu.CompilerParams(use_tc_tiling_on_sc=False),
    )
    def gather_kernel(idx_hbm, tab_hbm, out_hbm, idx_v, rows_v, g0, g1, o0, o1):
        wid = lax.axis_index("s") * NC + lax.axis_index("c")
        base_b = wid * per_wb
        pltpu.sync_copy(idx_hbm.at[wid], idx_v)

        def fire_gathers(t, s, sem):
            for i in range(K):
                pltpu.async_copy(
                    tab_hbm.at[idx_v.at[t * K + i]],
                    rows_v.at[s, pl.ds(i * CH, CH)],
                    sem,
                )

        def fire_out(t, s, sem):
            def one(b, carry):
                pltpu.async_copy(
                    rows_v.at[s, pl.ds(b * H, H)],
                    out_hbm.at[base_b + t * GB + b],
                    sem,
                )
                return carry
            lax.fori_loop(0, GB, one, 0)

        def drain_gathers(s, sem):
            pltpu.make_async_copy(tab_hbm.at[pl.ds(0, G)], rows_v.at[s], sem).wait()

        def drain_out(s, sem):
            def one(b, carry):
                pltpu.make_async_copy(
                    rows_v.at[s, pl.ds(0, H)], out_hbm.at[base_b], sem
                ).wait()
                return carry
            lax.fori_loop(0, GB, one, 0)

        fire_gathers(0, 0, g0)
        fire_gathers(1, 1, g1)

        def body(u, carry):
            t0 = 2 * u
            drain_gathers(0, g0)
            fire_out(t0 - 2, 0, o0)
            drain_out(0, o0)
            fire_gathers(t0, 0, g0)
            drain_gathers(1, g1)
            fire_out(t0 - 1, 1, o1)
            drain_out(1, o1)
            fire_gathers(t0 + 1, 1, g1)
            return carry

        lax.fori_loop(1, n_g // 2, body, 0)

        drain_gathers(0, g0)
        fire_out(n_g - 2, 0, o0)
        drain_gathers(1, g1)
        fire_out(n_g - 1, 1, o1)
        drain_out(0, o0)
        drain_out(1, o1)

    return gather_kernel(idx, weight)
